# baseline (speedup 1.0000x reference)
import functools

import jax
import jax.numpy as jnp
from jax import lax
from jax.experimental import pallas as pl
from jax.experimental.pallas import tpu as pltpu


def _topk_body(x1_ref, x2_ref, i3_ref, i4_ref, s1_ref, s2_ref, *, nsteps):
    k = pl.program_id(0)
    p1 = jnp.sum(x1_ref[...], axis=2)
    p2 = jnp.sum(x2_ref[...], axis=2)

    @pl.when(k == 0)
    def _init():
        s1_ref[...] = p1
        s2_ref[...] = p2

    @pl.when(k > 0)
    def _acc():
        s1_ref[...] += p1
        s2_ref[...] += p2

    @pl.when(k == nsteps - 1)
    def _rank():
        B, C = s1_ref.shape
        half = C // 2
        ii = lax.broadcasted_iota(jnp.int32, (B, C, C), 1)
        jj = lax.broadcasted_iota(jnp.int32, (B, C, C), 2)

        def perm_of(m):
            mi = m[:, :, None]
            mj = m[:, None, :]
            hit = (mj > mi) | ((mj == mi) & (jj < ii))
            rank = jnp.sum(hit.astype(jnp.int32), axis=2)
            er = rank[:, :, None] == jj
            perm = jnp.sum(jnp.where(er, ii, 0), axis=1)
            return perm[:, :half]

        i3_ref[...] = perm_of(s1_ref[...])
        i4_ref[...] = perm_of(s2_ref[...])


def _gather_body(idx_ref, x1_ref, x2_ref, out_ref):
    del idx_ref
    out_ref[0, 0, 0] = x1_ref[0, 0]
    out_ref[0, 1, 0] = x2_ref[0, 0]


def kernel(x1, x2):
    B, C, H, W = x1.shape
    half = C // 2
    HW = H * W
    chunk = 3584
    nsteps = HW // chunk

    x1r = x1.reshape(B, C, HW)
    x2r = x2.reshape(B, C, HW)

    i3, i4 = pl.pallas_call(
        functools.partial(_topk_body, nsteps=nsteps),
        grid=(nsteps,),
        in_specs=[
            pl.BlockSpec((B, C, chunk), lambda k: (0, 0, k)),
            pl.BlockSpec((B, C, chunk), lambda k: (0, 0, k)),
        ],
        out_specs=[
            pl.BlockSpec((B, half), lambda k: (0, 0)),
            pl.BlockSpec((B, half), lambda k: (0, 0)),
        ],
        out_shape=[
            jax.ShapeDtypeStruct((B, half), jnp.int32),
            jax.ShapeDtypeStruct((B, half), jnp.int32),
        ],
        scratch_shapes=[
            pltpu.VMEM((B, C), jnp.float32),
            pltpu.VMEM((B, C), jnp.float32),
        ],
    )(x1r, x2r)

    idx = jnp.stack((i3, i4), axis=1)

    grid_spec = pltpu.PrefetchScalarGridSpec(
        num_scalar_prefetch=1,
        grid=(B, half),
        in_specs=[
            pl.BlockSpec((1, 1, H, W), lambda b, c, idx: (b, idx[b, 0, c], 0, 0)),
            pl.BlockSpec((1, 1, H, W), lambda b, c, idx: (b, idx[b, 1, c], 0, 0)),
        ],
        out_specs=pl.BlockSpec(
            (1, 2, 1, H, W), lambda b, c, idx: (b, 0, c, 0, 0)
        ),
    )
    out = pl.pallas_call(
        _gather_body,
        grid_spec=grid_spec,
        out_shape=jax.ShapeDtypeStruct((B, 2, half, H, W), jnp.float32),
    )(idx, x1, x2)

    return out.reshape(B, C, H, W)


# speedup vs baseline: 1.2997x; 1.2997x over previous
import functools

import jax
import jax.numpy as jnp
from jax import lax
from jax.experimental import pallas as pl
from jax.experimental.pallas import tpu as pltpu


def _topk_body(x1_ref, x2_ref, i3_ref, i4_ref, s1_ref, s2_ref, *, nsteps):
    k = pl.program_id(0)
    p1 = jnp.sum(x1_ref[...], axis=(2, 3))
    p2 = jnp.sum(x2_ref[...], axis=(2, 3))

    @pl.when(k == 0)
    def _init():
        s1_ref[...] = p1
        s2_ref[...] = p2

    @pl.when(k > 0)
    def _acc():
        s1_ref[...] += p1
        s2_ref[...] += p2

    @pl.when(k == nsteps - 1)
    def _rank():
        B, C = s1_ref.shape
        half = C // 2
        ii = lax.broadcasted_iota(jnp.int32, (B, C, C), 1)
        jj = lax.broadcasted_iota(jnp.int32, (B, C, C), 2)

        def perm_of(m):
            mi = m[:, :, None]
            mj = m[:, None, :]
            hit = (mj > mi) | ((mj == mi) & (jj < ii))
            rank = jnp.sum(hit.astype(jnp.int32), axis=2)
            er = rank[:, :, None] == jj
            perm = jnp.sum(jnp.where(er, ii, 0), axis=1)
            return perm[:, :half]

        i3_ref[...] = perm_of(s1_ref[...])
        i4_ref[...] = perm_of(s2_ref[...])


def _gather_body(idx_ref, x1_ref, x2_ref, out_ref):
    del idx_ref
    out_ref[0, 0, 0] = x1_ref[0, 0]
    out_ref[0, 1, 0] = x2_ref[0, 0]


def kernel(x1, x2):
    B, C, H, W = x1.shape
    half = C // 2
    hchunk = 16
    nsteps = H // hchunk

    i3, i4 = pl.pallas_call(
        functools.partial(_topk_body, nsteps=nsteps),
        grid=(nsteps,),
        in_specs=[
            pl.BlockSpec((B, C, hchunk, W), lambda k: (0, 0, k, 0)),
            pl.BlockSpec((B, C, hchunk, W), lambda k: (0, 0, k, 0)),
        ],
        out_specs=[
            pl.BlockSpec((B, half), lambda k: (0, 0)),
            pl.BlockSpec((B, half), lambda k: (0, 0)),
        ],
        out_shape=[
            jax.ShapeDtypeStruct((B, half), jnp.int32),
            jax.ShapeDtypeStruct((B, half), jnp.int32),
        ],
        scratch_shapes=[
            pltpu.VMEM((B, C), jnp.float32),
            pltpu.VMEM((B, C), jnp.float32),
        ],
    )(x1, x2)

    idx = jnp.stack((i3, i4), axis=1)

    grid_spec = pltpu.PrefetchScalarGridSpec(
        num_scalar_prefetch=1,
        grid=(B, half),
        in_specs=[
            pl.BlockSpec((1, 1, H, W), lambda b, c, idx: (b, idx[b, 0, c], 0, 0)),
            pl.BlockSpec((1, 1, H, W), lambda b, c, idx: (b, idx[b, 1, c], 0, 0)),
        ],
        out_specs=pl.BlockSpec(
            (1, 2, 1, H, W), lambda b, c, idx: (b, 0, c, 0, 0)
        ),
    )
    out = pl.pallas_call(
        _gather_body,
        grid_spec=grid_spec,
        out_shape=jax.ShapeDtypeStruct((B, 2, half, H, W), jnp.float32),
    )(idx, x1, x2)

    return out.reshape(B, C, H, W)
